# baseline (speedup 1.0000x reference)
import functools

import jax
import jax.numpy as jnp
from jax import lax
from jax.experimental import pallas as pl
from jax.experimental.pallas import tpu as pltpu
from jax.experimental.pallas import tpu_sc as plsc

_EPS = 0.001


def _tc_body(cid_ref, mean_t_ref, std_t_ref, x_ref, o_ref, sc_ref, off_ref):
    @pl.when(pl.program_id(1) == 0)
    def _():
        c = cid_ref[pl.program_id(0)]
        srow = std_t_ref[pl.ds(c, 1), :]
        mrow = mean_t_ref[pl.ds(c, 1), :]
        sc = 1.0 / (jnp.exp(srow) + _EPS)
        sc_ref[...] = sc
        off_ref[...] = -mrow * sc

    o_ref[...] = x_ref[...] * sc_ref[...] + off_ref[...]


def kernel(x, context_id, initial_mean, initial_std):
    b, s, d = x.shape
    n_ctx = initial_mean.shape[0]
    cid = context_id[:, 0].astype(jnp.int32)

    bs = 256
    grid = (b, s // bs)
    out = pl.pallas_call(
        _tc_body,
        grid=grid,
        in_specs=[
            pl.BlockSpec(memory_space=pltpu.SMEM),
            pl.BlockSpec((n_ctx, d), lambda i, j: (0, 0)),
            pl.BlockSpec((n_ctx, d), lambda i, j: (0, 0)),
            pl.BlockSpec((1, bs, d), lambda i, j: (i, j, 0)),
        ],
        out_specs=pl.BlockSpec((1, bs, d), lambda i, j: (i, j, 0)),
        out_shape=jax.ShapeDtypeStruct((b, s, d), x.dtype),
        scratch_shapes=[
            pltpu.VMEM((1, d), jnp.float32),
            pltpu.VMEM((1, d), jnp.float32),
        ],
        compiler_params=pltpu.CompilerParams(
            dimension_semantics=("parallel", "arbitrary"),
        ),
    )(cid, initial_mean, initial_std, x)
    return out


# speedup vs baseline: 1.0139x; 1.0139x over previous
import functools

import jax
import jax.numpy as jnp
from jax import lax
from jax.experimental import pallas as pl
from jax.experimental.pallas import tpu as pltpu
from jax.experimental.pallas import tpu_sc as plsc

_EPS = 0.001


def _tc_body(cid_ref, mean_t_ref, std_t_ref, x_ref, o_ref, sc_ref, off_ref):
    @pl.when(pl.program_id(1) == 0)
    def _():
        c = cid_ref[pl.program_id(0)]
        srow = std_t_ref[pl.ds(c, 1), :]
        mrow = mean_t_ref[pl.ds(c, 1), :]
        sc = 1.0 / (jnp.exp(srow) + _EPS)
        sc_ref[...] = sc
        off_ref[...] = -mrow * sc

    o_ref[...] = x_ref[...] * sc_ref[...] + off_ref[...]


def kernel(x, context_id, initial_mean, initial_std):
    b, s, d = x.shape
    n_ctx = initial_mean.shape[0]
    cid = context_id[:, 0].astype(jnp.int32)

    bs = 512
    grid = (b, s // bs)
    out = pl.pallas_call(
        _tc_body,
        grid=grid,
        in_specs=[
            pl.BlockSpec(memory_space=pltpu.SMEM),
            pl.BlockSpec((n_ctx, d), lambda i, j: (0, 0)),
            pl.BlockSpec((n_ctx, d), lambda i, j: (0, 0)),
            pl.BlockSpec((1, bs, d), lambda i, j: (i, j, 0)),
        ],
        out_specs=pl.BlockSpec((1, bs, d), lambda i, j: (i, j, 0)),
        out_shape=jax.ShapeDtypeStruct((b, s, d), x.dtype),
        scratch_shapes=[
            pltpu.VMEM((1, d), jnp.float32),
            pltpu.VMEM((1, d), jnp.float32),
        ],
        compiler_params=pltpu.CompilerParams(
            dimension_semantics=("parallel", "arbitrary"),
        ),
    )(cid, initial_mean, initial_std, x)
    return out
